# Initial kernel scaffold; baseline (speedup 1.0000x reference)
import jax
import jax.numpy as jnp
from jax.experimental import pallas as pl


def kernel(x, edge_index, edge_attr, We, be, W1, b1, W2, b2, ln_g, ln_b, Wout, bout):
    raise NotImplementedError("write your pallas kernel here")


# speedup vs baseline: 2.4532x; 2.4532x over previous
import functools

import jax
import jax.numpy as jnp
from jax import lax
from jax.experimental import pallas as pl
from jax.experimental.pallas import tpu as pltpu
from jax.experimental.pallas import tpu_sc as plsc

N = 10000
E = 320000
D = 128
DE = 16
L = 3

NC = 2
NS = 16
NW = NC * NS
EPW = E // NW
C = 80
NCHUNK = EPW // C
ZRN = 200
NFL = N // ZRN
LANES = 16
G = D // LANES


def _agg_body(x_hbm, ee_hbm, src_hbm, dst_hbm, out_hbm,
              src_b, dst_b, ee_b, row_b, zbuf, acc, sem_e, sem_g):
    c = lax.axis_index("c")
    s = lax.axis_index("s")
    wid = c * NS + s

    def zrow(r, carry):
        for j in range(G):
            zbuf[r, pl.ds(j * LANES, LANES)] = jnp.zeros((LANES,), jnp.float32)
        return carry
    lax.fori_loop(0, ZRN, zrow, 0)
    for t in range((NFL + NS - 1) // NS):
        idx = s + NS * t

        @pl.when(idx < NFL)
        def _():
            pltpu.sync_copy(zbuf, acc.at[pl.ds(idx * ZRN, ZRN)])
    plsc.subcore_barrier()

    ebase = wid * EPW

    def chunk(k, carry):
        base = ebase + k * C
        pltpu.sync_copy(src_hbm.at[pl.ds(base, C)], src_b)
        pltpu.sync_copy(dst_hbm.at[pl.ds(base, C)], dst_b)
        cp_e = pltpu.async_copy(ee_hbm.at[pl.ds(base, C)], ee_b, sem_e)
        cp_g = pltpu.async_copy(x_hbm.at[src_b], row_b, sem_g)
        cp_e.wait()
        cp_g.wait()

        def edge(e, cin):
            for j in range(G):
                sl = pl.ds(j * LANES, LANES)
                v = row_b[e, sl] + ee_b[e, sl]
                ee_b[e, sl] = jnp.maximum(v, 0.0)
            return cin
        lax.fori_loop(0, C, edge, 0)
        pltpu.sync_copy(ee_b, acc.at[dst_b], add=True)
        return carry
    lax.fori_loop(0, NCHUNK, chunk, 0)

    plsc.subcore_barrier()
    for t in range((NFL + NS - 1) // NS):
        idx = s + NS * t

        @pl.when(idx < NFL)
        def _():
            off = idx * ZRN
            pltpu.sync_copy(acc.at[pl.ds(off, ZRN)],
                            out_hbm.at[c, pl.ds(off, ZRN)])


_agg = pl.kernel(
    _agg_body,
    out_type=jax.ShapeDtypeStruct((NC, N, D), jnp.float32),
    mesh=plsc.VectorSubcoreMesh(core_axis_name="c", subcore_axis_name="s",
                                num_cores=NC, num_subcores=NS),
    scratch_types=[
        pltpu.VMEM((C,), jnp.int32),
        pltpu.VMEM((C,), jnp.int32),
        pltpu.VMEM((C, D), jnp.float32),
        pltpu.VMEM((C, D), jnp.float32),
        pltpu.VMEM((ZRN, D), jnp.float32),
        pltpu.VMEM_SHARED((N, D), jnp.float32),
        pltpu.SemaphoreType.DMA,
        pltpu.SemaphoreType.DMA,
    ],
)


BE = 4000


def _ee_body(ea_ref, we_ref, be_ref, out_ref):
    r = jnp.dot(ea_ref[...], we_ref[0],
                preferred_element_type=jnp.float32) + be_ref[0]
    out_ref[...] = r[None]


_edge_proj = pl.pallas_call(
    _ee_body,
    grid=(L, E // BE),
    in_specs=[
        pl.BlockSpec((BE, DE), lambda l, e: (e, 0)),
        pl.BlockSpec((1, DE, D), lambda l, e: (l, 0, 0)),
        pl.BlockSpec((1, 1, D), lambda l, e: (l, 0, 0)),
    ],
    out_specs=pl.BlockSpec((1, BE, D), lambda l, e: (l, e, 0)),
    out_shape=jax.ShapeDtypeStruct((L, E, D), jnp.float32),
)


R = 2000


def _node_common(x_ref, a_ref, w1_ref, b1_ref, w2_ref, b2_ref, g_ref, bb_ref):
    x = x_ref[...]
    h = x + a_ref[0] + a_ref[1]
    t = jnp.maximum(
        jnp.dot(h, w1_ref[...], preferred_element_type=jnp.float32)
        + b1_ref[0], 0.0)
    t = jnp.dot(t, w2_ref[...], preferred_element_type=jnp.float32) + b2_ref[0]
    mu = jnp.mean(t, axis=-1, keepdims=True)
    var = jnp.mean((t - mu) ** 2, axis=-1, keepdims=True)
    t = (t - mu) / jnp.sqrt(var + 1e-5) * g_ref[0] + bb_ref[0]
    return jax.nn.gelu(t) + x


def _node_mid_body(x_ref, a_ref, w1_ref, b1_ref, w2_ref, b2_ref, g_ref,
                   bb_ref, out_ref):
    out_ref[...] = _node_common(x_ref, a_ref, w1_ref, b1_ref, w2_ref, b2_ref,
                                g_ref, bb_ref)


def _node_last_body(x_ref, a_ref, w1_ref, b1_ref, w2_ref, b2_ref, g_ref,
                    bb_ref, wo_ref, bo_ref, out_ref):
    y = _node_common(x_ref, a_ref, w1_ref, b1_ref, w2_ref, b2_ref,
                     g_ref, bb_ref)
    out_ref[...] = jnp.dot(y, wo_ref[...],
                           preferred_element_type=jnp.float32) + bo_ref[0]


_NODE_SPECS = [
    pl.BlockSpec((R, D), lambda i: (i, 0)),
    pl.BlockSpec((NC, R, D), lambda i: (0, i, 0)),
    pl.BlockSpec((D, D), lambda i: (0, 0)),
    pl.BlockSpec((1, D), lambda i: (0, 0)),
    pl.BlockSpec((D, D), lambda i: (0, 0)),
    pl.BlockSpec((1, D), lambda i: (0, 0)),
    pl.BlockSpec((1, D), lambda i: (0, 0)),
    pl.BlockSpec((1, D), lambda i: (0, 0)),
]

_node_mid = pl.pallas_call(
    _node_mid_body,
    grid=(N // R,),
    in_specs=_NODE_SPECS,
    out_specs=pl.BlockSpec((R, D), lambda i: (i, 0)),
    out_shape=jax.ShapeDtypeStruct((N, D), jnp.float32),
)

_node_last = pl.pallas_call(
    _node_last_body,
    grid=(N // R,),
    in_specs=_NODE_SPECS + [
        pl.BlockSpec((D, D), lambda i: (0, 0)),
        pl.BlockSpec((1, D), lambda i: (0, 0)),
    ],
    out_specs=pl.BlockSpec((R, D), lambda i: (i, 0)),
    out_shape=jax.ShapeDtypeStruct((N, D), jnp.float32),
)


def kernel(x, edge_index, edge_attr, We, be, W1, b1, W2, b2, ln_g, ln_b,
           Wout, bout):
    src = edge_index[0]
    dst = edge_index[1]
    ee = _edge_proj(edge_attr, We, be[:, None])
    for i in range(L):
        agg2 = _agg(x, ee[i], src, dst)
        args = (x, agg2, W1[i], b1[i][None], W2[i], b2[i][None],
                ln_g[i][None], ln_b[i][None])
        if i < L - 1:
            x = _node_mid(*args)
        else:
            x = _node_last(*args, Wout, bout[None])
    return x
